# baseline (speedup 1.0000x reference)
import functools

import jax
import jax.numpy as jnp
from jax import lax
from jax.experimental import pallas as pl
from jax.experimental.pallas import tpu as pltpu
from jax.experimental.pallas import tpu_sc as plsc

_N = 10000
_NUM_HE = 2000
_NP = 10240
_HEP = 2048
_CHUNK = 128
_NC, _NS = 2, 16
_D = 128


def _ceil_to(x, m):
    return (x + m - 1) // m * m


def _sc_scatter_pass(table, src, dst, zeros, *, na_pad, ch):
    mesh = plsc.VectorSubcoreMesh(core_axis_name="c", subcore_axis_name="s")

    ch2 = ch // 2

    def body(table_h, src_h, dst_h, zeros_h, out_h, src_v, dst_v, rows_a,
             rows_b, acc_sh, sem_a, sem_b, sem_sa, sem_sb):
        c = lax.axis_index("c")
        s = lax.axis_index("s")

        @pl.when(s == 0)
        def _():
            pltpu.sync_copy(zeros_h, acc_sh)

        plsc.subcore_barrier()

        for h in range(2):
            pltpu.sync_copy(src_h.at[c].at[s].at[h], src_v)
            pltpu.sync_copy(dst_h.at[c].at[s].at[h], dst_v)
            pltpu.async_copy(table_h.at[src_v.at[0]], rows_a, sem_a)
            pltpu.async_copy(table_h.at[src_v.at[1]], rows_b, sem_b)

            def step(i, carry):
                j0 = 2 * i
                j1 = j0 + 1
                pltpu.make_async_copy(table_h.at[src_v.at[j0]], rows_a,
                                      sem_a).wait()
                pltpu.async_copy(rows_a, acc_sh.at[dst_v.at[j0]], sem_sa,
                                 add=True)
                pltpu.make_async_copy(table_h.at[src_v.at[j1]], rows_b,
                                      sem_b).wait()
                pltpu.async_copy(rows_b, acc_sh.at[dst_v.at[j1]], sem_sb,
                                 add=True)
                pltpu.make_async_copy(rows_a, acc_sh.at[dst_v.at[j0]],
                                      sem_sa).wait()

                @pl.when(i + 1 < ch2 // 2)
                def _():
                    pltpu.async_copy(table_h.at[src_v.at[j0 + 2]], rows_a,
                                     sem_a)

                pltpu.make_async_copy(rows_b, acc_sh.at[dst_v.at[j1]],
                                      sem_sb).wait()

                @pl.when(i + 1 < ch2 // 2)
                def _():
                    pltpu.async_copy(table_h.at[src_v.at[j1 + 2]], rows_b,
                                     sem_b)

                return carry

            lax.fori_loop(0, ch2 // 2, step, 0)

        plsc.subcore_barrier()
        rp = na_pad // _NS
        pltpu.sync_copy(acc_sh.at[pl.ds(s * rp, rp)],
                        out_h.at[c].at[pl.ds(s * rp, rp)])

    kern = pl.kernel(
        body,
        out_type=jax.ShapeDtypeStruct((_NC, na_pad, _D), jnp.float32),
        mesh=mesh,
        scratch_types=[
            pltpu.VMEM((ch2, _CHUNK), jnp.int32),
            pltpu.VMEM((ch2, _CHUNK), jnp.int32),
            pltpu.VMEM((_CHUNK, _D), jnp.float32),
            pltpu.VMEM((_CHUNK, _D), jnp.float32),
            pltpu.VMEM_SHARED((na_pad, _D), jnp.float32),
            pltpu.SemaphoreType.DMA,
            pltpu.SemaphoreType.DMA,
            pltpu.SemaphoreType.DMA,
            pltpu.SemaphoreType.DMA,
        ],
    )
    return kern(table, src, dst, zeros)


def _sc_count(cidx, ones_h, zeros, *, ra, ch):
    mesh = plsc.VectorSubcoreMesh(core_axis_name="c", subcore_axis_name="s")
    ch2 = ch // 2

    def body(cidx_h, ones_hh, zeros_h, out_h, idx_v, ones_v, acc_sh, sem):
        c = lax.axis_index("c")
        s = lax.axis_index("s")
        pltpu.sync_copy(ones_hh, ones_v)

        @pl.when(s == 0)
        def _():
            pltpu.sync_copy(zeros_h, acc_sh)

        plsc.subcore_barrier()

        for h in range(2):
            pltpu.sync_copy(cidx_h.at[c].at[s].at[h], idx_v)

            def fire(i, carry):
                pltpu.async_copy(ones_v, acc_sh.at[idx_v.at[i]], sem,
                                 add=True)
                return carry

            lax.fori_loop(0, ch2, fire, 0)

            def drain(i, carry):
                pltpu.make_async_copy(ones_v, acc_sh.at[idx_v.at[i]],
                                      sem).wait()
                return carry

            lax.fori_loop(0, ch2, drain, 0)

        plsc.subcore_barrier()
        rp = ra // _NS
        pltpu.sync_copy(acc_sh.at[pl.ds(s * rp, rp)],
                        out_h.at[c].at[pl.ds(s * rp, rp)])

    kern = pl.kernel(
        body,
        out_type=jax.ShapeDtypeStruct((_NC, ra, _D), jnp.float32),
        mesh=mesh,
        scratch_types=[
            pltpu.VMEM((ch2, _CHUNK), jnp.int32),
            pltpu.VMEM((_CHUNK, _D), jnp.float32),
            pltpu.VMEM_SHARED((ra, _D), jnp.float32),
            pltpu.SemaphoreType.DMA,
        ],
    )
    return kern(cidx, ones_h, zeros)


_BN = 512
_NBLK = _NP // _BN
_CNT_E_IDX = 5


def _deg_dinv(cnt_blk):
    deg = 1.0 + cnt_blk[0, :, 0] + cnt_blk[1, :, 0]
    return lax.rsqrt(deg)


def _mm0_body(x_ref, cnt_ref, o_ref):
    dinv = _deg_dinv(cnt_ref[...])
    o_ref[...] = x_ref[...] * dinv[:, None]


def _tc_mm0(x_np, cnts):
    return pl.pallas_call(
        _mm0_body,
        grid=(_NBLK,),
        in_specs=[
            pl.BlockSpec((_BN, 128), lambda i: (i, 0)),
            pl.BlockSpec((2, _BN, _D), lambda i: (0, i, 0)),
        ],
        out_specs=pl.BlockSpec((_BN, 128), lambda i: (i, 0)),
        out_shape=jax.ShapeDtypeStruct((_NP, 128), jnp.float32),
    )(x_np, cnts)


def _mm2_body(acc_ref, xd_ref, cnt_ref, w1_ref, b_ref, w_ref, o_ref):
    dinv = _deg_dinv(cnt_ref[...])
    s = acc_ref[0] + acc_ref[1] + xd_ref[...]
    h1 = jnp.dot(s, w1_ref[...], preferred_element_type=jnp.float32)
    pre = h1 * dinv[:, None] + b_ref[0][None, :]
    o1 = jnp.maximum(pre, 0.0)
    g2 = jnp.dot(o1, w_ref[...], preferred_element_type=jnp.float32)
    o_ref[...] = g2 * dinv[:, None]


def _tc_mm2(acc1, xd, cnts, w_g1, b_g1, w_g2):
    return pl.pallas_call(
        _mm2_body,
        grid=(_NBLK,),
        in_specs=[
            pl.BlockSpec((2, _BN, 128), lambda i: (0, i, 0)),
            pl.BlockSpec((_BN, 128), lambda i: (i, 0)),
            pl.BlockSpec((2, _BN, _D), lambda i: (0, i, 0)),
            pl.BlockSpec((128, 256), lambda i: (0, 0)),
            pl.BlockSpec((1, 256), lambda i: (0, 0)),
            pl.BlockSpec((256, 128), lambda i: (0, 0)),
        ],
        out_specs=pl.BlockSpec((_BN, 128), lambda i: (i, 0)),
        out_shape=jax.ShapeDtypeStruct((_NP, 128), jnp.float32),
    )(acc1, xd, cnts, w_g1, b_g1.reshape(1, 256), w_g2)


def _mm3_body(acc_ref, g_ref, cnt_ref, b_ref, xh_ref, we_ref, be_ref,
              n_ref, m_ref):
    dinv = _deg_dinv(cnt_ref[...])
    n_ref[...] = ((acc_ref[0] + acc_ref[1] + g_ref[...]) * dinv[:, None]
                  + b_ref[0][None, :])
    m_ref[...] = jnp.dot(xh_ref[...], we_ref[...],
                         preferred_element_type=jnp.float32) + be_ref[0][None, :]


def _tc_mm3(acc2, g2, cnts, b_g2, x_hnp, we0, be0):
    return pl.pallas_call(
        _mm3_body,
        grid=(_NBLK,),
        in_specs=[
            pl.BlockSpec((2, _BN, 128), lambda i: (0, i, 0)),
            pl.BlockSpec((_BN, 128), lambda i: (i, 0)),
            pl.BlockSpec((2, _BN, _D), lambda i: (0, i, 0)),
            pl.BlockSpec((1, 128), lambda i: (0, 0)),
            pl.BlockSpec((_BN, 128), lambda i: (i, 0)),
            pl.BlockSpec((128, 128), lambda i: (0, 0)),
            pl.BlockSpec((1, 128), lambda i: (0, 0)),
        ],
        out_specs=[
            pl.BlockSpec((_BN, 128), lambda i: (i, 0)),
            pl.BlockSpec((_BN, 128), lambda i: (i, 0)),
        ],
        out_shape=[
            jax.ShapeDtypeStruct((_NP, 128), jnp.float32),
            jax.ShapeDtypeStruct((_NP, 128), jnp.float32),
        ],
    )(acc2, g2, cnts, b_g2.reshape(1, 128), x_hnp, we0, be0.reshape(1, 128))


def _e_body(es_ref, cnt_ref, wn_ref, t_ref, e_ref):
    ce = jnp.maximum(cnt_ref[0, :, 0] + cnt_ref[1, :, 0], 1.0)
    e = (es_ref[0] + es_ref[1]) / ce[:, None]
    t_ref[...] = jnp.dot(e, wn_ref[...], preferred_element_type=jnp.float32)
    if e_ref is not None:
        e_ref[...] = e


def _tc_e(es, cnts, wn, want_e):
    body = _e_body if want_e else functools.partial(_e_body, e_ref=None)
    out_specs = [pl.BlockSpec((_HEP, 128), lambda i: (0, 0))]
    out_shape = [jax.ShapeDtypeStruct((_HEP, 128), jnp.float32)]
    if want_e:
        out_specs.append(pl.BlockSpec((_HEP, 128), lambda i: (0, 0)))
        out_shape.append(jax.ShapeDtypeStruct((_HEP, 128), jnp.float32))
    res = pl.pallas_call(
        body,
        grid=(1,),
        in_specs=[
            pl.BlockSpec((2, _HEP, 128), lambda i: (0, 0, 0)),
            pl.BlockSpec((2, _HEP, _D), lambda i: (0, _CNT_E_IDX, 0)),
            pl.BlockSpec((128, 128), lambda i: (0, 0)),
        ],
        out_specs=out_specs,
        out_shape=out_shape,
    )(es, cnts, wn)
    return res if want_e else res[0]


def _x_body(xs_ref, cnt_ref, m_ref, wn_ref, bn_ref, a_ref, we_ref, be_ref,
            o_ref, last):
    cn = cnt_ref[0, :, 0] + cnt_ref[1, :, 0] + 1.0
    t_self = jnp.dot(m_ref[...], wn_ref[...],
                     preferred_element_type=jnp.float32)
    x = (xs_ref[0] + xs_ref[1] + t_self) / cn[:, None] + bn_ref[0][None, :]
    a = a_ref[0, 0]
    x = jnp.where(x > 0, x, a * x)
    if last:
        o_ref[...] = x
    else:
        o_ref[...] = jnp.dot(x, we_ref[...],
                             preferred_element_type=jnp.float32) + be_ref[0][None, :]


def _tc_x(xs, cnts, m, wn, bn, a, we=None, be=None):
    last = we is None
    if last:
        we = wn
        be = bn
    body = functools.partial(_x_body, last=last)
    return pl.pallas_call(
        body,
        grid=(_NBLK,),
        in_specs=[
            pl.BlockSpec((2, _BN, 128), lambda i: (0, i, 0)),
            pl.BlockSpec((2, _BN, _D), lambda i: (0, i, 0)),
            pl.BlockSpec((_BN, 128), lambda i: (i, 0)),
            pl.BlockSpec((128, 128), lambda i: (0, 0)),
            pl.BlockSpec((1, 128), lambda i: (0, 0)),
            pl.BlockSpec((1, 1), lambda i: (0, 0)),
            pl.BlockSpec((128, 128), lambda i: (0, 0)),
            pl.BlockSpec((1, 128), lambda i: (0, 0)),
        ],
        out_specs=pl.BlockSpec((_BN, 128), lambda i: (i, 0)),
        out_shape=jax.ShapeDtypeStruct((_NP, 128), jnp.float32),
    )(xs, cnts, m, wn, bn.reshape(1, 128), a.reshape(1, 1), we,
      be.reshape(1, 128))


def _shard(ids, total, pad):
    npad = total - ids.shape[0]
    v = jnp.concatenate([ids, pad[:npad]], 0)
    return v.reshape(_NC, _NS, 2, -1, _CHUNK)


def kernel(x_n, x_hn, edge_index, hyperedge_index, num_nodes, num_edges,
           W_g1, b_g1, W_g2, b_g2, We0, be0, Wn0, bn0, We1, be1, Wn1, bn1,
           prelu_a):
    n = x_n.shape[0]
    e = edge_index.shape[1]
    hnnz = hyperedge_index.shape[1]

    ep = _ceil_to(e, _NC * _NS * _CHUNK * 4)
    hp = _ceil_to(hnnz, _NC * _NS * _CHUNK * 4)
    ch_e = ep // (_NC * _NS * _CHUNK)
    ch_h = hp // (_NC * _NS * _CHUNK)

    x_np = jnp.pad(x_n, ((0, _NP - n), (0, 0)))
    x_hnp = jnp.pad(x_hn, ((0, _NP - n), (0, 0)))

    pad_iota = jnp.arange(max(ep - e, hp - hnnz), dtype=jnp.int32)
    pad_n_src = pad_iota % n
    pad_n_dst = n + pad_iota % (_NP - n)
    pad_he_src = pad_iota % _NUM_HE
    pad_he_dst = _NUM_HE + pad_iota % (_HEP - _NUM_HE)

    src_g = _shard(edge_index[0], ep, pad_n_src)
    dst_g = _shard(edge_index[1], ep, pad_n_dst)
    hn = hyperedge_index[0]
    he = hyperedge_index[1]
    hn_src = _shard(hn, hp, pad_n_src)
    hn_dst = _shard(hn, hp, pad_n_dst)
    he_src = _shard(he, hp, pad_he_src)
    he_dst = _shard(he, hp, pad_he_dst)

    ra = _NP + _HEP
    ctot = e + hnnz
    ctot_p = _ceil_to(ctot, _NC * _NS * _CHUNK * 2)
    cpad = n + jnp.arange(ctot_p - ctot, dtype=jnp.int32) % (_NP - n)
    cidx_a = jnp.concatenate([edge_index[1], he + _NP, cpad], 0)
    cidx_a = cidx_a.reshape(_NC, _NS, 2, -1, _CHUNK)
    ch_a = 2 * cidx_a.shape[3]

    ones_h = jnp.ones((_CHUNK, _D), jnp.float32)
    zeros_a = jnp.zeros((ra, _D), jnp.float32)
    zeros_n = jnp.zeros((_NP, _D), jnp.float32)
    zeros_he = jnp.zeros((_HEP, _D), jnp.float32)

    cnt_a = _sc_count(cidx_a, ones_h, zeros_a, ra=ra, ch=ch_a)
    cnt_b = _sc_count(hn_dst, ones_h, zeros_n, ra=_NP, ch=ch_h)

    xd = _tc_mm0(x_np, cnt_a)
    acc1 = _sc_scatter_pass(xd, src_g, dst_g, zeros_n, na_pad=_NP, ch=ch_e)
    g2 = _tc_mm2(acc1, xd, cnt_a, W_g1, b_g1, W_g2)
    acc2 = _sc_scatter_pass(g2, src_g, dst_g, zeros_n, na_pad=_NP, ch=ch_e)
    n_out, m0 = _tc_mm3(acc2, g2, cnt_a, b_g2, x_hnp, We0, be0)

    es0 = _sc_scatter_pass(m0, hn_src, he_dst, zeros_he,
                           na_pad=_HEP, ch=ch_h)
    t0h = _tc_e(es0, cnt_a, Wn0, want_e=False)
    xs0 = _sc_scatter_pass(t0h, he_src, hn_dst, zeros_n,
                           na_pad=_NP, ch=ch_h)
    m1 = _tc_x(xs0, cnt_b, m0, Wn0, bn0, prelu_a, We1, be1)

    es1 = _sc_scatter_pass(m1, hn_src, he_dst, zeros_he,
                           na_pad=_HEP, ch=ch_h)
    t1h, e_full = _tc_e(es1, cnt_a, Wn1, want_e=True)
    xs1 = _sc_scatter_pass(t1h, he_src, hn_dst, zeros_n,
                           na_pad=_NP, ch=ch_h)
    x1 = _tc_x(xs1, cnt_b, m1, Wn1, bn1, prelu_a)

    return (n_out[:n], x1[:n], e_full[:_NUM_HE])


# speedup vs baseline: 1.2848x; 1.2848x over previous
import functools

import jax
import jax.numpy as jnp
from jax import lax
from jax.experimental import pallas as pl
from jax.experimental.pallas import tpu as pltpu
from jax.experimental.pallas import tpu_sc as plsc

_N = 10000
_NUM_HE = 2000
_NP = 10240
_HEP = 2048
_CHUNK = 128
_NC, _NS = 2, 16
_D = 128


def _ceil_to(x, m):
    return (x + m - 1) // m * m


def _sc_scatter_pass(table, src, dst, zeros, *, na_pad, ch):
    mesh = plsc.VectorSubcoreMesh(core_axis_name="c", subcore_axis_name="s")

    ch2 = ch // 2

    def body(table_h, src_h, dst_h, zeros_h, out_h, src_v, dst_v, rows_a,
             rows_b, acc_sh, sem_a, sem_b, sem_sa, sem_sb):
        c = lax.axis_index("c")
        s = lax.axis_index("s")

        @pl.when(s == 0)
        def _():
            pltpu.sync_copy(zeros_h, acc_sh)

        plsc.subcore_barrier()

        for h in range(2):
            pltpu.sync_copy(src_h.at[c].at[s].at[h], src_v)
            pltpu.sync_copy(dst_h.at[c].at[s].at[h], dst_v)
            pltpu.async_copy(table_h.at[src_v.at[0]], rows_a, sem_a)

            def step(i, carry):
                j0 = 2 * i
                j1 = j0 + 1
                pltpu.make_async_copy(table_h.at[src_v.at[j0]], rows_a,
                                      sem_a).wait()
                pltpu.async_copy(table_h.at[src_v.at[j1]], rows_b, sem_b)
                pltpu.sync_copy(rows_a, acc_sh.at[dst_v.at[j0]], add=True)
                pltpu.make_async_copy(table_h.at[src_v.at[j1]], rows_b,
                                      sem_b).wait()

                @pl.when(i + 1 < ch2 // 2)
                def _():
                    pltpu.async_copy(table_h.at[src_v.at[j0 + 2]], rows_a,
                                     sem_a)

                pltpu.sync_copy(rows_b, acc_sh.at[dst_v.at[j1]], add=True)
                return carry

            lax.fori_loop(0, ch2 // 2, step, 0)

        plsc.subcore_barrier()
        rp = na_pad // _NS
        pltpu.sync_copy(acc_sh.at[pl.ds(s * rp, rp)],
                        out_h.at[c].at[pl.ds(s * rp, rp)])

    kern = pl.kernel(
        body,
        out_type=jax.ShapeDtypeStruct((_NC, na_pad, _D), jnp.float32),
        mesh=mesh,
        scratch_types=[
            pltpu.VMEM((ch2, _CHUNK), jnp.int32),
            pltpu.VMEM((ch2, _CHUNK), jnp.int32),
            pltpu.VMEM((_CHUNK, _D), jnp.float32),
            pltpu.VMEM((_CHUNK, _D), jnp.float32),
            pltpu.VMEM_SHARED((na_pad, _D), jnp.float32),
            pltpu.SemaphoreType.DMA,
            pltpu.SemaphoreType.DMA,
            pltpu.SemaphoreType.DMA,
            pltpu.SemaphoreType.DMA,
        ],
    )
    return kern(table, src, dst, zeros)


_RA = 2 * _NP + _HEP


def _sc_count(cidx):
    mesh = plsc.VectorSubcoreMesh(core_axis_name="c", subcore_axis_name="s")
    ept = cidx.shape[2]

    def body(cidx_h, out_h, idx_v, hist_v):
        c = lax.axis_index("c")
        s = lax.axis_index("s")

        def zero(i, carry):
            hist_v[pl.ds(i * 16, 16)] = jnp.zeros((16,), jnp.float32)
            return carry

        lax.fori_loop(0, _RA // 16, zero, 0)
        pltpu.sync_copy(cidx_h.at[c].at[s], idx_v)
        ones = jnp.ones((16,), jnp.float32)

        def step(i, carry):
            iv = idx_v[pl.ds(i * 16, 16)]
            plsc.addupdate_scatter(hist_v, [iv], ones)
            return carry

        lax.fori_loop(0, ept // 16, step, 0)
        pltpu.sync_copy(hist_v, out_h.at[c].at[s])

    kern = pl.kernel(
        body,
        out_type=jax.ShapeDtypeStruct((_NC, _NS, _RA), jnp.float32),
        mesh=mesh,
        compiler_params=pltpu.CompilerParams(needs_layout_passes=False),
        scratch_types=[
            pltpu.VMEM((ept,), jnp.int32),
            pltpu.VMEM((_RA,), jnp.float32),
        ],
    )
    return kern(cidx)


_BN = 512
_NBLK = _NP // _BN
_CNT_N_BLK = _NP // _BN
_CNT_E_IDX = 10


def _cnt_sum(cnt_blk):
    return jnp.sum(cnt_blk, axis=(0, 1))


def _deg_dinv(cnt_blk):
    return lax.rsqrt(1.0 + _cnt_sum(cnt_blk))


def _mm0_body(x_ref, cnt_ref, o_ref):
    dinv = _deg_dinv(cnt_ref[...])
    o_ref[...] = x_ref[...] * dinv[:, None]


def _tc_mm0(x_np, cnts):
    return pl.pallas_call(
        _mm0_body,
        grid=(_NBLK,),
        in_specs=[
            pl.BlockSpec((_BN, 128), lambda i: (i, 0)),
            pl.BlockSpec((2, _NS, _BN), lambda i: (0, 0, i)),
        ],
        out_specs=pl.BlockSpec((_BN, 128), lambda i: (i, 0)),
        out_shape=jax.ShapeDtypeStruct((_NP, 128), jnp.float32),
    )(x_np, cnts)


def _mm2_body(acc_ref, xd_ref, cnt_ref, w1_ref, b_ref, w_ref, o_ref):
    dinv = _deg_dinv(cnt_ref[...])
    s = acc_ref[0] + acc_ref[1] + xd_ref[...]
    h1 = jnp.dot(s, w1_ref[...], preferred_element_type=jnp.float32)
    pre = h1 * dinv[:, None] + b_ref[0][None, :]
    o1 = jnp.maximum(pre, 0.0)
    g2 = jnp.dot(o1, w_ref[...], preferred_element_type=jnp.float32)
    o_ref[...] = g2 * dinv[:, None]


def _tc_mm2(acc1, xd, cnts, w_g1, b_g1, w_g2):
    return pl.pallas_call(
        _mm2_body,
        grid=(_NBLK,),
        in_specs=[
            pl.BlockSpec((2, _BN, 128), lambda i: (0, i, 0)),
            pl.BlockSpec((_BN, 128), lambda i: (i, 0)),
            pl.BlockSpec((2, _NS, _BN), lambda i: (0, 0, i)),
            pl.BlockSpec((128, 256), lambda i: (0, 0)),
            pl.BlockSpec((1, 256), lambda i: (0, 0)),
            pl.BlockSpec((256, 128), lambda i: (0, 0)),
        ],
        out_specs=pl.BlockSpec((_BN, 128), lambda i: (i, 0)),
        out_shape=jax.ShapeDtypeStruct((_NP, 128), jnp.float32),
    )(acc1, xd, cnts, w_g1, b_g1.reshape(1, 256), w_g2)


def _mm3_body(acc_ref, g_ref, cnt_ref, b_ref, xh_ref, we_ref, be_ref,
              n_ref, m_ref):
    dinv = _deg_dinv(cnt_ref[...])
    n_ref[...] = ((acc_ref[0] + acc_ref[1] + g_ref[...]) * dinv[:, None]
                  + b_ref[0][None, :])
    m_ref[...] = jnp.dot(xh_ref[...], we_ref[...],
                         preferred_element_type=jnp.float32) + be_ref[0][None, :]


def _tc_mm3(acc2, g2, cnts, b_g2, x_hnp, we0, be0):
    return pl.pallas_call(
        _mm3_body,
        grid=(_NBLK,),
        in_specs=[
            pl.BlockSpec((2, _BN, 128), lambda i: (0, i, 0)),
            pl.BlockSpec((_BN, 128), lambda i: (i, 0)),
            pl.BlockSpec((2, _NS, _BN), lambda i: (0, 0, i)),
            pl.BlockSpec((1, 128), lambda i: (0, 0)),
            pl.BlockSpec((_BN, 128), lambda i: (i, 0)),
            pl.BlockSpec((128, 128), lambda i: (0, 0)),
            pl.BlockSpec((1, 128), lambda i: (0, 0)),
        ],
        out_specs=[
            pl.BlockSpec((_BN, 128), lambda i: (i, 0)),
            pl.BlockSpec((_BN, 128), lambda i: (i, 0)),
        ],
        out_shape=[
            jax.ShapeDtypeStruct((_NP, 128), jnp.float32),
            jax.ShapeDtypeStruct((_NP, 128), jnp.float32),
        ],
    )(acc2, g2, cnts, b_g2.reshape(1, 128), x_hnp, we0, be0.reshape(1, 128))


def _e_body(es_ref, cnt_ref, wn_ref, t_ref, e_ref):
    ce = jnp.maximum(_cnt_sum(cnt_ref[...]), 1.0)
    e = (es_ref[0] + es_ref[1]) / ce[:, None]
    t_ref[...] = jnp.dot(e, wn_ref[...], preferred_element_type=jnp.float32)
    if e_ref is not None:
        e_ref[...] = e


def _tc_e(es, cnts, wn, want_e):
    body = _e_body if want_e else functools.partial(_e_body, e_ref=None)
    out_specs = [pl.BlockSpec((_HEP, 128), lambda i: (0, 0))]
    out_shape = [jax.ShapeDtypeStruct((_HEP, 128), jnp.float32)]
    if want_e:
        out_specs.append(pl.BlockSpec((_HEP, 128), lambda i: (0, 0)))
        out_shape.append(jax.ShapeDtypeStruct((_HEP, 128), jnp.float32))
    res = pl.pallas_call(
        body,
        grid=(1,),
        in_specs=[
            pl.BlockSpec((2, _HEP, 128), lambda i: (0, 0, 0)),
            pl.BlockSpec((2, _NS, _HEP), lambda i: (0, 0, _CNT_E_IDX)),
            pl.BlockSpec((128, 128), lambda i: (0, 0)),
        ],
        out_specs=out_specs,
        out_shape=out_shape,
    )(es, cnts, wn)
    return res if want_e else res[0]


def _x_body(xs_ref, cnt_ref, m_ref, wn_ref, bn_ref, a_ref, we_ref, be_ref,
            o_ref, last):
    cn = _cnt_sum(cnt_ref[...]) + 1.0
    t_self = jnp.dot(m_ref[...], wn_ref[...],
                     preferred_element_type=jnp.float32)
    x = (xs_ref[0] + xs_ref[1] + t_self) / cn[:, None] + bn_ref[0][None, :]
    a = a_ref[0, 0]
    x = jnp.where(x > 0, x, a * x)
    if last:
        o_ref[...] = x
    else:
        o_ref[...] = jnp.dot(x, we_ref[...],
                             preferred_element_type=jnp.float32) + be_ref[0][None, :]


def _tc_x(xs, cnts, m, wn, bn, a, we=None, be=None):
    last = we is None
    if last:
        we = wn
        be = bn
    body = functools.partial(_x_body, last=last)
    return pl.pallas_call(
        body,
        grid=(_NBLK,),
        in_specs=[
            pl.BlockSpec((2, _BN, 128), lambda i: (0, i, 0)),
            pl.BlockSpec((2, _NS, _BN), lambda i: (0, 0, _CNT_N_BLK + i)),
            pl.BlockSpec((_BN, 128), lambda i: (i, 0)),
            pl.BlockSpec((128, 128), lambda i: (0, 0)),
            pl.BlockSpec((1, 128), lambda i: (0, 0)),
            pl.BlockSpec((1, 1), lambda i: (0, 0)),
            pl.BlockSpec((128, 128), lambda i: (0, 0)),
            pl.BlockSpec((1, 128), lambda i: (0, 0)),
        ],
        out_specs=pl.BlockSpec((_BN, 128), lambda i: (i, 0)),
        out_shape=jax.ShapeDtypeStruct((_NP, 128), jnp.float32),
    )(xs, cnts, m, wn, bn.reshape(1, 128), a.reshape(1, 1), we,
      be.reshape(1, 128))


def _shard(ids, total, pad):
    npad = total - ids.shape[0]
    v = jnp.concatenate([ids, pad[:npad]], 0)
    return v.reshape(_NC, _NS, 2, -1, _CHUNK)


def kernel(x_n, x_hn, edge_index, hyperedge_index, num_nodes, num_edges,
           W_g1, b_g1, W_g2, b_g2, We0, be0, Wn0, bn0, We1, be1, Wn1, bn1,
           prelu_a):
    n = x_n.shape[0]
    e = edge_index.shape[1]
    hnnz = hyperedge_index.shape[1]

    ep = _ceil_to(e, _NC * _NS * _CHUNK * 4)
    hp = _ceil_to(hnnz, _NC * _NS * _CHUNK * 4)
    ch_e = ep // (_NC * _NS * _CHUNK)
    ch_h = hp // (_NC * _NS * _CHUNK)

    x_np = jnp.pad(x_n, ((0, _NP - n), (0, 0)))
    x_hnp = jnp.pad(x_hn, ((0, _NP - n), (0, 0)))

    pad_iota = jnp.arange(max(ep - e, hp - hnnz), dtype=jnp.int32)
    pad_n_src = pad_iota % n
    pad_n_dst = n + pad_iota % (_NP - n)
    pad_he_src = pad_iota % _NUM_HE
    pad_he_dst = _NUM_HE + pad_iota % (_HEP - _NUM_HE)

    src_g = _shard(edge_index[0], ep, pad_n_src)
    dst_g = _shard(edge_index[1], ep, pad_n_dst)
    hn = hyperedge_index[0]
    he = hyperedge_index[1]
    hn_src = _shard(hn, hp, pad_n_src)
    hn_dst = _shard(hn, hp, pad_n_dst)
    he_src = _shard(he, hp, pad_he_src)
    he_dst = _shard(he, hp, pad_he_dst)

    ctot = e + 2 * hnnz
    ctot_p = _ceil_to(ctot, _NC * _NS * 16)
    cpad = n + jnp.arange(ctot_p - ctot, dtype=jnp.int32) % (_NP - n)
    cidx = jnp.concatenate(
        [edge_index[1], hn + _NP, he + 2 * _NP, cpad], 0)
    cnts = _sc_count(cidx.reshape(_NC, _NS, -1))

    zeros_n = jnp.zeros((_NP, _D), jnp.float32)
    zeros_he = jnp.zeros((_HEP, _D), jnp.float32)

    xd = _tc_mm0(x_np, cnts)
    acc1 = _sc_scatter_pass(xd, src_g, dst_g, zeros_n, na_pad=_NP, ch=ch_e)
    g2 = _tc_mm2(acc1, xd, cnts, W_g1, b_g1, W_g2)
    acc2 = _sc_scatter_pass(g2, src_g, dst_g, zeros_n, na_pad=_NP, ch=ch_e)
    n_out, m0 = _tc_mm3(acc2, g2, cnts, b_g2, x_hnp, We0, be0)

    es0 = _sc_scatter_pass(m0, hn_src, he_dst, zeros_he,
                           na_pad=_HEP, ch=ch_h)
    t0h = _tc_e(es0, cnts, Wn0, want_e=False)
    xs0 = _sc_scatter_pass(t0h, he_src, hn_dst, zeros_n,
                           na_pad=_NP, ch=ch_h)
    m1 = _tc_x(xs0, cnts, m0, Wn0, bn0, prelu_a, We1, be1)

    es1 = _sc_scatter_pass(m1, hn_src, he_dst, zeros_he,
                           na_pad=_HEP, ch=ch_h)
    t1h, e_full = _tc_e(es1, cnts, Wn1, want_e=True)
    xs1 = _sc_scatter_pass(t1h, he_src, hn_dst, zeros_n,
                           na_pad=_NP, ch=ch_h)
    x1 = _tc_x(xs1, cnts, m1, Wn1, bn1, prelu_a)

    return (n_out[:n], x1[:n], e_full[:_NUM_HE])
